# Initial kernel scaffold; baseline (speedup 1.0000x reference)
import jax
import jax.numpy as jnp
from jax.experimental import pallas as pl


def kernel(x, edge_index, adj_vals, W, prelu_a):
    raise NotImplementedError("write your pallas kernel here")


# speedup vs baseline: 6.6843x; 6.6843x over previous
import functools

import jax
import jax.numpy as jnp
from jax import lax
from jax.experimental import pallas as pl
from jax.experimental.pallas import tpu as pltpu
from jax.experimental.pallas import tpu_sc as plsc

N_NODES = 10000
D = 128
E = 320000
L = 16
NC = 2
NS = 16
NW = NC * NS
E_PER = 10080
E_PAD = NW * E_PER
K = 80
NCHUNK = E_PER // K
PAIRS = NCHUNK // 2
N_ACC = 10240
ROWS_PER_TILE = N_ACC // NS
ZROWS = 128


def _mm_body(x_ref, w_ref, o_ref):
    o_ref[...] = jnp.dot(x_ref[...], w_ref[...], preferred_element_type=jnp.float32)


def _matmul(x, W):
    M = x.shape[0]
    BM = 1000
    return pl.pallas_call(
        _mm_body,
        grid=(M // BM,),
        in_specs=[
            pl.BlockSpec((BM, D), lambda i: (i, 0)),
            pl.BlockSpec((D, D), lambda i: (0, 0)),
        ],
        out_specs=pl.BlockSpec((BM, D), lambda i: (i, 0)),
        out_shape=jax.ShapeDtypeStruct((M, D), jnp.float32),
    )(x, W)


def _fin_body(p_ref, a_ref, o_ref):
    s = p_ref[0] + p_ref[1]
    a = a_ref[0]
    o_ref[...] = jnp.where(s >= 0.0, s, a * s)


def _finish(partials, prelu_a):
    BM = 1000
    return pl.pallas_call(
        _fin_body,
        grid=(N_NODES // BM,),
        in_specs=[
            pl.BlockSpec((NC, BM, D), lambda i: (0, i, 0)),
            pl.BlockSpec(memory_space=pltpu.SMEM),
        ],
        out_specs=pl.BlockSpec((BM, D), lambda i: (i, 0)),
        out_shape=jax.ShapeDtypeStruct((N_NODES, D), jnp.float32),
    )(partials, prelu_a)


def _lane_splat(v, lane):
    return lax.gather(
        v,
        jnp.full((L, 1), lane, jnp.int32),
        dimension_numbers=lax.GatherDimensionNumbers(
            offset_dims=(), collapsed_slice_dims=(0,), start_index_map=(0,)),
        slice_sizes=(1,),
        mode=lax.GatherScatterMode.PROMISE_IN_BOUNDS,
    )


def _sc_spmm(h, row3, col1, vals1):
    mesh = plsc.VectorSubcoreMesh(core_axis_name="c", subcore_axis_name="s")

    @functools.partial(
        pl.kernel,
        mesh=mesh,
        out_type=jax.ShapeDtypeStruct((NC, N_ACC, D), jnp.float32),
        scratch_types=[
            pltpu.VMEM((E_PER,), jnp.int32),
            pltpu.VMEM((NCHUNK, K), jnp.int32),
            pltpu.VMEM((K,), jnp.float32),
            pltpu.VMEM((K,), jnp.float32),
            pltpu.VMEM((K, D), jnp.float32),
            pltpu.VMEM((K, D), jnp.float32),
            pltpu.VMEM_SHARED((N_ACC, D), jnp.float32),
            pltpu.SemaphoreType.DMA,
            pltpu.SemaphoreType.DMA,
        ],
    )
    def spmm(h_hbm, row_hbm, col_hbm, vals_hbm, out_hbm,
             col_v, row2_v, vbuf0, vbuf1, buf0, buf1, acc, sem0, sem1):
        c = lax.axis_index("c")
        s = lax.axis_index("s")
        wid = c * NS + s
        ebase = wid * E_PER

        pltpu.sync_copy(col_hbm.at[pl.ds(ebase, E_PER)], col_v)
        pltpu.sync_copy(row_hbm.at[wid], row2_v)

        zero16 = jnp.zeros((L,), jnp.float32)

        def zrow(i, carry):
            for j in range(D // L):
                buf0[i, pl.ds(j * L, L)] = zero16
            return carry

        lax.fori_loop(0, K, zrow, 0)
        for z in range(ROWS_PER_TILE // K):
            pltpu.sync_copy(
                buf0, acc.at[pl.ds(s * ROWS_PER_TILE + z * K, K)])
        plsc.subcore_barrier()

        def gather_start(ci, buf, vbuf, sem):
            pltpu.async_copy(h_hbm.at[col_v.at[pl.ds(ci * K, K)]], buf, sem)
            pltpu.async_copy(vals_hbm.at[pl.ds(ebase + ci * K, K)], vbuf, sem)

        def gather_wait(ci, buf, vbuf, sem):
            pltpu.make_async_copy(
                h_hbm.at[col_v.at[pl.ds(ci * K, K)]], buf, sem).wait()
            pltpu.make_async_copy(
                vals_hbm.at[pl.ds(ebase + ci * K, K)], vbuf, sem).wait()

        def scale(buf, vbuf):
            def grp(g, carry):
                vv = vbuf[pl.ds(g * L, L)]
                for lane in range(L):
                    sp = _lane_splat(vv, lane)
                    e = g * L + lane
                    for j in range(D // L):
                        buf[e, pl.ds(j * L, L)] = buf[e, pl.ds(j * L, L)] * sp
                return carry
            lax.fori_loop(0, K // L, grp, 0)

        def process(ci, buf, vbuf):
            scale(buf, vbuf)
            pltpu.sync_copy(buf, acc.at[row2_v.at[ci]], add=True)

        gather_start(0, buf0, vbuf0, sem0)

        def pair(p, carry):
            c0 = 2 * p
            gather_start(c0 + 1, buf1, vbuf1, sem1)
            gather_wait(c0, buf0, vbuf0, sem0)
            process(c0, buf0, vbuf0)

            @pl.when(p < PAIRS - 1)
            def _():
                gather_start(c0 + 2, buf0, vbuf0, sem0)

            gather_wait(c0 + 1, buf1, vbuf1, sem1)
            process(c0 + 1, buf1, vbuf1)
            return carry

        lax.fori_loop(0, PAIRS, pair, 0)

        plsc.subcore_barrier()
        for z in range(ROWS_PER_TILE // ZROWS):
            base = s * ROWS_PER_TILE + z * ZROWS
            pltpu.sync_copy(acc.at[pl.ds(base, ZROWS)],
                            out_hbm.at[c, pl.ds(base, ZROWS)])

    return spmm(h, row3, col1, vals1)


def kernel(x, edge_index, adj_vals, W, prelu_a):
    h = _matmul(x, W)
    row = edge_index[0].astype(jnp.int32)
    col = edge_index[1].astype(jnp.int32)
    pad = E_PAD - E
    row3 = jnp.pad(row, (0, pad)).reshape(NW, NCHUNK, K)
    col1 = jnp.pad(col, (0, pad))
    vals1 = jnp.pad(adj_vals, (0, pad))
    partials = _sc_spmm(h, row3, col1, vals1)[:, :N_NODES]
    a = jnp.reshape(prelu_a, (1,)).astype(jnp.float32)
    return _finish(partials, a)
